# baseline (speedup 1.0000x reference)
import functools

import jax
import jax.numpy as jnp
from jax import lax
from jax.experimental import pallas as pl
from jax.experimental.pallas import tpu as pltpu
from jax.experimental.pallas import tpu_sc as plsc

B = 16384
ED = 32
HL = 64
G = 20
NC, NS = 2, 16
NW = NC * NS
BPW = B // NW
CHUNK = 128
NCH = BPW // CHUNK

@functools.cache
def _sc_gather_fn():
    mesh = plsc.VectorSubcoreMesh(core_axis_name="c", subcore_axis_name="s",
                                  num_cores=NC, num_subcores=NS)

    @functools.partial(
        pl.kernel,
        out_type=(
            jax.ShapeDtypeStruct((B, ED), jnp.float32),
            jax.ShapeDtypeStruct((B, ED), jnp.float32),
            jax.ShapeDtypeStruct((B,), jnp.float32),
            jax.ShapeDtypeStruct((B,), jnp.float32),
        ),
        mesh=mesh,
        compiler_params=pltpu.CompilerParams(use_tc_tiling_on_sc=False),
        scratch_types=(
            pltpu.VMEM((NCH, CHUNK), jnp.int32),
            pltpu.VMEM((NCH, CHUNK), jnp.int32),
            pltpu.VMEM((BPW, ED), jnp.float32),
            pltpu.VMEM((BPW, ED), jnp.float32),
            pltpu.VMEM((BPW,), jnp.float32),
            pltpu.VMEM((BPW,), jnp.float32),
            pltpu.SemaphoreType.DMA,
        ),
    )
    def _sc_gather(uidx_hbm, midx_hbm, uemb_hbm, memb_hbm, ubias_hbm, mbias_hbm,
                   urows_out, mrows_out, ub_out, mb_out,
                   uidx_v, midx_v, urows_v, mrows_v, ub_v, mb_v, sem):
        wid = lax.axis_index("s") * NC + lax.axis_index("c")
        base = wid * BPW
        pltpu.sync_copy(uidx_hbm.at[wid], uidx_v)
        pltpu.sync_copy(midx_hbm.at[wid], midx_v)
        copies = []
        for j in range(NCH):
            sl = pl.ds(j * CHUNK, CHUNK)
            copies.append(pltpu.async_copy(uemb_hbm.at[uidx_v.at[j]], urows_v.at[sl], sem))
            copies.append(pltpu.async_copy(memb_hbm.at[midx_v.at[j]], mrows_v.at[sl], sem))
            copies.append(pltpu.async_copy(ubias_hbm.at[uidx_v.at[j]], ub_v.at[sl], sem))
            copies.append(pltpu.async_copy(mbias_hbm.at[midx_v.at[j]], mb_v.at[sl], sem))
        for c in copies:
            c.wait()
        pltpu.sync_copy(urows_v, urows_out.at[pl.ds(base, BPW)])
        pltpu.sync_copy(mrows_v, mrows_out.at[pl.ds(base, BPW)])
        pltpu.sync_copy(ub_v, ub_out.at[pl.ds(base, BPW)])
        pltpu.sync_copy(mb_v, mb_out.at[pl.ds(base, BPW)])

    return _sc_gather


def _tc_body(u_ref, m_ref, g_ref, ub_ref, mb_ref, gW_ref, gb_ref,
             w1u_ref, w1m_ref, w1g_ref, b1_ref, w2_ref, c2_ref, out_ref):
    cdims = (((1,), (1,)), ((), ()))
    u = u_ref[...]
    m = m_ref[...]
    ge = lax.dot_general(g_ref[...], gW_ref[...], cdims,
                         preferred_element_type=jnp.float32) + gb_ref[...]
    acc = lax.dot_general(u, w1u_ref[...], cdims, preferred_element_type=jnp.float32)
    acc = acc + lax.dot_general(m, w1m_ref[...], cdims, preferred_element_type=jnp.float32)
    acc = acc + lax.dot_general(ge, w1g_ref[...], cdims, preferred_element_type=jnp.float32)
    h = jnp.maximum(acc + b1_ref[...], 0.0)
    mlp = lax.dot_general(h, w2_ref[...], cdims, preferred_element_type=jnp.float32)
    dp = jnp.sum(u * m, axis=1)
    out_ref[...] = dp + mlp[:, 0] + ub_ref[...] + mb_ref[...] + c2_ref[0, 0]


BLK = 2048


def _tc_forward(urows, mrows, genre, ub, mb, gW, gb2, w1u, w1m, w1g, b12, W2, c2):
    return pl.pallas_call(
        _tc_body,
        grid=(B // BLK,),
        in_specs=[
            pl.BlockSpec((BLK, ED), lambda i: (i, 0)),
            pl.BlockSpec((BLK, ED), lambda i: (i, 0)),
            pl.BlockSpec((BLK, G), lambda i: (i, 0)),
            pl.BlockSpec((BLK,), lambda i: (i,)),
            pl.BlockSpec((BLK,), lambda i: (i,)),
            pl.BlockSpec((ED, G), lambda i: (0, 0)),
            pl.BlockSpec((1, ED), lambda i: (0, 0)),
            pl.BlockSpec((HL, ED), lambda i: (0, 0)),
            pl.BlockSpec((HL, ED), lambda i: (0, 0)),
            pl.BlockSpec((HL, ED), lambda i: (0, 0)),
            pl.BlockSpec((1, HL), lambda i: (0, 0)),
            pl.BlockSpec((1, HL), lambda i: (0, 0)),
            pl.BlockSpec((1, 1), lambda i: (0, 0)),
        ],
        out_specs=pl.BlockSpec((BLK,), lambda i: (i,)),
        out_shape=jax.ShapeDtypeStruct((B,), jnp.float32),
    )(urows, mrows, genre, ub, mb, gW, gb2, w1u, w1m, w1g, b12, W2, c2)


def kernel(userIndices, movieIndices, genreIndeces, userEmb, movieEmb,
           userBiasT, movieBiasT, bias, gW, gb, W1, b1, W2, b2):
    uidx = userIndices.astype(jnp.int32).reshape(NW, NCH, CHUNK)
    midx = movieIndices.astype(jnp.int32).reshape(NW, NCH, CHUNK)
    urows, mrows, ub, mb = _sc_gather_fn()(
        uidx, midx, userEmb, movieEmb, userBiasT[:, 0], movieBiasT[:, 0])
    w1u = W1[:, :ED]
    w1m = W1[:, ED:2 * ED]
    w1g = W1[:, 2 * ED:]
    c2 = (bias + b2).reshape(1, 1)
    return _tc_forward(urows, mrows, genreIndeces, ub, mb, gW,
                       gb.reshape(1, ED), w1u, w1m, w1g,
                       b1.reshape(1, HL), W2, c2)
